# baseline (speedup 1.0000x reference)
import functools

import jax
import jax.numpy as jnp
from jax import lax
from jax.experimental import pallas as pl
from jax.experimental.pallas import tpu as pltpu
from jax.experimental.pallas import tpu_sc as plsc

_N = 10000
_E = 320000
_F_IN = 128
_HID = 64
_NCLS = 2
_L = 3
_NG = 64

_SC_CORES = 2
_SC_TILES = 16
_NW = _SC_CORES * _SC_TILES
_EW = _E // _NW
_NPAD = 10240
_RPT = _NPAD // _SC_TILES
_CHUNK = 100
_IT = _EW // _CHUNK


def _make_agg(F, nbuf):
  chunk, it = _CHUNK, _IT
  assert it % nbuf == 0
  mesh = plsc.VectorSubcoreMesh(core_axis_name="c", subcore_axis_name="s")

  @functools.partial(
      pl.kernel,
      out_type=jax.ShapeDtypeStruct((_SC_CORES, _NPAD, F), jnp.float32),
      mesh=mesh,
      compiler_params=pltpu.CompilerParams(use_tc_tiling_on_sc=False),
      scratch_types=(
          [pltpu.VMEM((it, chunk), jnp.int32)] * 2
          + [pltpu.VMEM((chunk, F), jnp.float32)] * nbuf
          + [pltpu.VMEM_SHARED((_NPAD, F), jnp.float32)]
          + [pltpu.SemaphoreType.DMA] * nbuf
      ),
  )
  def agg(h_hbm, edge_hbm, zeros_hbm, out_hbm, src_v, dst_v, *rest):
    rows = rest[:nbuf]
    acc_sh = rest[nbuf]
    sems = rest[nbuf + 1:]
    c = lax.axis_index("c")
    s = lax.axis_index("s")
    w = c * _SC_TILES + s
    pltpu.sync_copy(zeros_hbm.at[pl.ds(s * _RPT, _RPT)],
                    acc_sh.at[pl.ds(s * _RPT, _RPT)])
    pltpu.sync_copy(edge_hbm.at[0, pl.ds(w * it, it)], src_v)
    pltpu.sync_copy(edge_hbm.at[1, pl.ds(w * it, it)], dst_v)
    plsc.subcore_barrier()

    for b in range(nbuf):
      pltpu.async_copy(h_hbm.at[src_v.at[b]], rows[b], sems[b])

    def body(j, carry):
      for k in range(nbuf):
        i = nbuf * j + k
        pltpu.make_async_copy(h_hbm.at[src_v.at[0]], rows[k], sems[k]).wait()
        pltpu.sync_copy(rows[k], acc_sh.at[dst_v.at[i]], add=True)
        nxt = lax.rem(i + nbuf, it)
        pltpu.async_copy(h_hbm.at[src_v.at[nxt]], rows[k], sems[k])
      return carry

    lax.fori_loop(0, it // nbuf, body, 0)
    for b in range(nbuf):
      pltpu.make_async_copy(h_hbm.at[src_v.at[0]], rows[b], sems[b]).wait()
    plsc.subcore_barrier()
    pltpu.sync_copy(acc_sh.at[pl.ds(s * _RPT, _RPT)],
                    out_hbm.at[c, pl.ds(s * _RPT, _RPT)])

  return agg


_agg128 = _make_agg(_F_IN, 2)
_agg64 = _make_agg(_HID, 5)


def _mlp(h, a0, a1, w1, b1, w2, b2, g, be):
  z = h + a0 + a1
  z = jnp.dot(z, w1, preferred_element_type=jnp.float32) + b1
  z = jnp.maximum(z, 0.0)
  z = jnp.dot(z, w2, preferred_element_type=jnp.float32) + b2
  mean = jnp.mean(z, axis=0, keepdims=True)
  zc = z - mean
  var = jnp.mean(zc * zc, axis=0, keepdims=True)
  zn = zc / jnp.sqrt(var + 1e-5)
  return jnp.maximum(zn * g + be, 0.0)


def _mlp_body(h_ref, a_ref, w1_ref, b1_ref, w2_ref, b2_ref, g_ref, be_ref,
              o_ref):
  o_ref[...] = _mlp(h_ref[...], a_ref[0, :_N], a_ref[1, :_N], w1_ref[...],
                    b1_ref[...], w2_ref[...], b2_ref[...], g_ref[...],
                    be_ref[...])


def _mlp_call(h, agg, w1, b1, w2, b2, gamma, beta):
  return pl.pallas_call(
      _mlp_body,
      out_shape=jax.ShapeDtypeStruct((_N, _HID), jnp.float32),
  )(h, agg, w1, b1.reshape(1, -1), w2, b2.reshape(1, -1),
    gamma.reshape(1, -1), beta.reshape(1, -1))


def _last_body(h2_ref, a_ref, w1_ref, b1_ref, w2_ref, b2_ref, g_ref, be_ref,
               h1_ref, b2d_ref, wj_ref, bj_ref, wc1_ref, bc1_ref,
               wc2_ref, bc2_ref, o_ref):
  h2 = h2_ref[...]
  h3 = _mlp(h2, a_ref[0, :_N], a_ref[1, :_N], w1_ref[...],
            b1_ref[...], w2_ref[...], b2_ref[...], g_ref[...], be_ref[...])
  hc = jnp.concatenate([h1_ref[...], h2, h3], axis=1)
  hj = jnp.dot(hc, wj_ref[...], preferred_element_type=jnp.float32) + bj_ref[...]
  gids = lax.broadcasted_iota(jnp.int32, (_NG, _N), 0)
  mask = (gids == b2d_ref[...]).astype(jnp.float32)
  pooled = jnp.dot(mask, hj, preferred_element_type=jnp.float32,
                   precision=lax.Precision.HIGHEST)
  cmid = jnp.maximum(
      jnp.dot(pooled, wc1_ref[...], preferred_element_type=jnp.float32)
      + bc1_ref[...], 0.0)
  o_ref[...] = (jnp.dot(cmid, wc2_ref[...], preferred_element_type=jnp.float32)
                + bc2_ref[...])


def _last_call(h2, agg, w1, b1, w2, b2, gamma, beta, h1, batch2d, wj, bj,
               wc1, bc1, wc2, bc2):
  return pl.pallas_call(
      _last_body,
      out_shape=jax.ShapeDtypeStruct((_NG, _NCLS), jnp.float32),
  )(h2, agg, w1, b1.reshape(1, -1), w2, b2.reshape(1, -1),
    gamma.reshape(1, -1), beta.reshape(1, -1), h1, batch2d, wj,
    bj.reshape(1, -1), wc1, bc1.reshape(1, -1), wc2, bc2.reshape(1, -1))


def kernel(x, edge_index, batch, params):
  edge3d = edge_index.reshape(2, _E // _CHUNK, _CHUNK)
  batch2d = batch.reshape(1, _N)
  zeros128 = jnp.zeros((_NPAD, _F_IN), jnp.float32)
  zeros64 = jnp.zeros((_NPAD, _HID), jnp.float32)

  agg = _agg128(x, edge3d, zeros128)
  h1 = _mlp_call(x, agg, params['W1_0'], params['b1_0'], params['W2_0'],
                 params['b2_0'], params['gamma_0'], params['beta_0'])
  agg = _agg64(h1, edge3d, zeros64)
  h2 = _mlp_call(h1, agg, params['W1_1'], params['b1_1'], params['W2_1'],
                 params['b2_1'], params['gamma_1'], params['beta_1'])
  agg = _agg64(h2, edge3d, zeros64)
  return _last_call(h2, agg, params['W1_2'], params['b1_2'], params['W2_2'],
                    params['b2_2'], params['gamma_2'], params['beta_2'],
                    h1, batch2d, params['Wj'], params['bj'],
                    params['Wc1'], params['bc1'], params['Wc2'],
                    params['bc2'])


# speedup vs baseline: 1.0204x; 1.0204x over previous
import functools

import jax
import jax.numpy as jnp
from jax import lax
from jax.experimental import pallas as pl
from jax.experimental.pallas import tpu as pltpu
from jax.experimental.pallas import tpu_sc as plsc

_N = 10000
_E = 320000
_F_IN = 128
_HID = 64
_NCLS = 2
_L = 3
_NG = 64

_SC_CORES = 2
_SC_TILES = 16
_NW = _SC_CORES * _SC_TILES
_EW = _E // _NW
_NPAD = 10240
_RPT = _NPAD // _SC_TILES
_CHUNK = 80
_IT = _EW // _CHUNK


def _make_agg(F, nbuf):
  chunk, it = _CHUNK, _IT
  full, tail = it // nbuf, it % nbuf
  mesh = plsc.VectorSubcoreMesh(core_axis_name="c", subcore_axis_name="s")

  @functools.partial(
      pl.kernel,
      out_type=jax.ShapeDtypeStruct((_SC_CORES, _NPAD, F), jnp.float32),
      mesh=mesh,
      compiler_params=pltpu.CompilerParams(use_tc_tiling_on_sc=False),
      scratch_types=(
          [pltpu.VMEM((_EW,), jnp.int32)] * 2
          + [pltpu.VMEM((chunk, F), jnp.float32)] * nbuf
          + [pltpu.VMEM_SHARED((_NPAD, F), jnp.float32)]
          + [pltpu.SemaphoreType.DMA] * nbuf
      ),
  )
  def agg(h_hbm, edge_hbm, zeros_hbm, out_hbm, src_v, dst_v, *rest):
    rows = rest[:nbuf]
    acc_sh = rest[nbuf]
    sems = rest[nbuf + 1:]
    c = lax.axis_index("c")
    s = lax.axis_index("s")
    w = c * _SC_TILES + s
    pltpu.sync_copy(zeros_hbm.at[pl.ds(s * _RPT, _RPT)],
                    acc_sh.at[pl.ds(s * _RPT, _RPT)])
    pltpu.sync_copy(edge_hbm.at[0, pl.ds(w * _EW, _EW)], src_v)
    pltpu.sync_copy(edge_hbm.at[1, pl.ds(w * _EW, _EW)], dst_v)
    plsc.subcore_barrier()

    def sidx(i):
      return src_v.at[pl.ds(i * chunk, chunk)]

    def didx(i):
      return dst_v.at[pl.ds(i * chunk, chunk)]

    for b in range(nbuf):
      pltpu.async_copy(h_hbm.at[sidx(b)], rows[b], sems[b])

    def body(j, carry):
      for k in range(nbuf):
        i = nbuf * j + k
        pltpu.make_async_copy(h_hbm.at[sidx(0)], rows[k], sems[k]).wait()
        pltpu.sync_copy(rows[k], acc_sh.at[didx(i)], add=True)
        nxt = lax.rem(i + nbuf, it)
        pltpu.async_copy(h_hbm.at[sidx(nxt)], rows[k], sems[k])
      return carry

    lax.fori_loop(0, full, body, 0)
    for k in range(nbuf):
      pltpu.make_async_copy(h_hbm.at[sidx(0)], rows[k], sems[k]).wait()
      if k < tail:
        pltpu.sync_copy(rows[k], acc_sh.at[didx(full * nbuf + k)], add=True)
    plsc.subcore_barrier()
    pltpu.sync_copy(acc_sh.at[pl.ds(s * _RPT, _RPT)],
                    out_hbm.at[c, pl.ds(s * _RPT, _RPT)])

  return agg


_agg128 = _make_agg(_F_IN, 2)
_agg64 = _make_agg(_HID, 5)


def _mlp(h, a0, a1, w1, b1, w2, b2, g, be):
  z = h + a0 + a1
  z = jnp.dot(z, w1, preferred_element_type=jnp.float32) + b1
  z = jnp.maximum(z, 0.0)
  z = jnp.dot(z, w2, preferred_element_type=jnp.float32) + b2
  mean = jnp.mean(z, axis=0, keepdims=True)
  zc = z - mean
  var = jnp.mean(zc * zc, axis=0, keepdims=True)
  zn = zc / jnp.sqrt(var + 1e-5)
  return jnp.maximum(zn * g + be, 0.0)


def _mlp_body(h_ref, a_ref, w1_ref, b1_ref, w2_ref, b2_ref, g_ref, be_ref,
              o_ref):
  o_ref[...] = _mlp(h_ref[...], a_ref[0, :_N], a_ref[1, :_N], w1_ref[...],
                    b1_ref[...], w2_ref[...], b2_ref[...], g_ref[...],
                    be_ref[...])


def _mlp_call(h, agg, w1, b1, w2, b2, gamma, beta):
  return pl.pallas_call(
      _mlp_body,
      out_shape=jax.ShapeDtypeStruct((_N, _HID), jnp.float32),
  )(h, agg, w1, b1.reshape(1, -1), w2, b2.reshape(1, -1),
    gamma.reshape(1, -1), beta.reshape(1, -1))


def _last_body(h2_ref, a_ref, w1_ref, b1_ref, w2_ref, b2_ref, g_ref, be_ref,
               h1_ref, b2d_ref, wj_ref, bj_ref, wc1_ref, bc1_ref,
               wc2_ref, bc2_ref, o_ref):
  h2 = h2_ref[...]
  h3 = _mlp(h2, a_ref[0, :_N], a_ref[1, :_N], w1_ref[...],
            b1_ref[...], w2_ref[...], b2_ref[...], g_ref[...], be_ref[...])
  hc = jnp.concatenate([h1_ref[...], h2, h3], axis=1)
  hj = jnp.dot(hc, wj_ref[...], preferred_element_type=jnp.float32) + bj_ref[...]
  gids = lax.broadcasted_iota(jnp.int32, (_NG, _N), 0)
  mask = (gids == b2d_ref[...]).astype(jnp.float32)
  pooled = jnp.dot(mask, hj, preferred_element_type=jnp.float32,
                   precision=lax.Precision.HIGHEST)
  cmid = jnp.maximum(
      jnp.dot(pooled, wc1_ref[...], preferred_element_type=jnp.float32)
      + bc1_ref[...], 0.0)
  o_ref[...] = (jnp.dot(cmid, wc2_ref[...], preferred_element_type=jnp.float32)
                + bc2_ref[...])


def _last_call(h2, agg, w1, b1, w2, b2, gamma, beta, h1, batch2d, wj, bj,
               wc1, bc1, wc2, bc2):
  return pl.pallas_call(
      _last_body,
      out_shape=jax.ShapeDtypeStruct((_NG, _NCLS), jnp.float32),
  )(h2, agg, w1, b1.reshape(1, -1), w2, b2.reshape(1, -1),
    gamma.reshape(1, -1), beta.reshape(1, -1), h1, batch2d, wj,
    bj.reshape(1, -1), wc1, bc1.reshape(1, -1), wc2, bc2.reshape(1, -1))


def kernel(x, edge_index, batch, params):
  batch2d = batch.reshape(1, _N)
  zeros128 = jnp.zeros((_NPAD, _F_IN), jnp.float32)
  zeros64 = jnp.zeros((_NPAD, _HID), jnp.float32)

  agg = _agg128(x, edge_index, zeros128)
  h1 = _mlp_call(x, agg, params['W1_0'], params['b1_0'], params['W2_0'],
                 params['b2_0'], params['gamma_0'], params['beta_0'])
  agg = _agg64(h1, edge_index, zeros64)
  h2 = _mlp_call(h1, agg, params['W1_1'], params['b1_1'], params['W2_1'],
                 params['b2_1'], params['gamma_1'], params['beta_1'])
  agg = _agg64(h2, edge_index, zeros64)
  return _last_call(h2, agg, params['W1_2'], params['b1_2'], params['W2_2'],
                    params['b2_2'], params['gamma_2'], params['beta_2'],
                    h1, batch2d, params['Wj'], params['bj'],
                    params['Wc1'], params['bc1'], params['Wc2'],
                    params['bc2'])


# speedup vs baseline: 1.0344x; 1.0137x over previous
import functools

import jax
import jax.numpy as jnp
from jax import lax
from jax.experimental import pallas as pl
from jax.experimental.pallas import tpu as pltpu
from jax.experimental.pallas import tpu_sc as plsc

_N = 10000
_E = 320000
_F_IN = 128
_HID = 64
_NCLS = 2
_L = 3
_NG = 64

_SC_CORES = 2
_SC_TILES = 16
_NW = _SC_CORES * _SC_TILES
_EW = _E // _NW
_NPAD = 10240
_RPT = _NPAD // _SC_TILES
_CHUNK = 80
_IT = _EW // _CHUNK


def _make_agg(F, nbuf):
  chunk, it = _CHUNK, _IT
  full, tail = it // nbuf, it % nbuf
  mesh = plsc.VectorSubcoreMesh(core_axis_name="c", subcore_axis_name="s")

  @functools.partial(
      pl.kernel,
      out_type=jax.ShapeDtypeStruct((_SC_CORES, _NPAD, F), jnp.float32),
      mesh=mesh,
      compiler_params=pltpu.CompilerParams(use_tc_tiling_on_sc=False),
      scratch_types=(
          [pltpu.VMEM((_EW,), jnp.int32)] * 2
          + [pltpu.VMEM((chunk, F), jnp.float32)] * nbuf
          + [pltpu.VMEM_SHARED((_NPAD, F), jnp.float32)]
          + [pltpu.SemaphoreType.DMA] * nbuf
      ),
  )
  def agg(h_hbm, edge_hbm, out_hbm, src_v, dst_v, *rest):
    rows = rest[:nbuf]
    acc_sh = rest[nbuf]
    sems = rest[nbuf + 1:]
    c = lax.axis_index("c")
    s = lax.axis_index("s")
    w = c * _SC_TILES + s
    last = _N - (_SC_TILES - 1) * _RPT

    @pl.when(s != _SC_TILES - 1)
    def _():
      pltpu.sync_copy(h_hbm.at[pl.ds(s * _RPT, _RPT)],
                      acc_sh.at[pl.ds(s * _RPT, _RPT)])

    @pl.when(s == _SC_TILES - 1)
    def _():
      pltpu.sync_copy(h_hbm.at[pl.ds((_SC_TILES - 1) * _RPT, last)],
                      acc_sh.at[pl.ds((_SC_TILES - 1) * _RPT, last)])
    pltpu.sync_copy(edge_hbm.at[0, pl.ds(w * _EW, _EW)], src_v)
    pltpu.sync_copy(edge_hbm.at[1, pl.ds(w * _EW, _EW)], dst_v)
    plsc.subcore_barrier()

    def sidx(i):
      return src_v.at[pl.ds(i * chunk, chunk)]

    def didx(i):
      return dst_v.at[pl.ds(i * chunk, chunk)]

    for b in range(nbuf):
      pltpu.async_copy(h_hbm.at[sidx(b)], rows[b], sems[b])

    def body(j, carry):
      for k in range(nbuf):
        i = nbuf * j + k
        pltpu.make_async_copy(h_hbm.at[sidx(0)], rows[k], sems[k]).wait()
        pltpu.sync_copy(rows[k], acc_sh.at[didx(i)], add=True)
        nxt = lax.rem(i + nbuf, it)
        pltpu.async_copy(h_hbm.at[sidx(nxt)], rows[k], sems[k])
      return carry

    lax.fori_loop(0, full, body, 0)
    for k in range(nbuf):
      pltpu.make_async_copy(h_hbm.at[sidx(0)], rows[k], sems[k]).wait()
      if k < tail:
        pltpu.sync_copy(rows[k], acc_sh.at[didx(full * nbuf + k)], add=True)
    plsc.subcore_barrier()
    pltpu.sync_copy(acc_sh.at[pl.ds(s * _RPT, _RPT)],
                    out_hbm.at[c, pl.ds(s * _RPT, _RPT)])

  return agg


_agg128 = _make_agg(_F_IN, 2)
_agg64 = _make_agg(_HID, 5)


def _mlp(h, a0, a1, w1, b1, w2, b2, g, be):
  z = a0 + a1 - h
  z = jnp.dot(z, w1, preferred_element_type=jnp.float32) + b1
  z = jnp.maximum(z, 0.0)
  z = jnp.dot(z, w2, preferred_element_type=jnp.float32) + b2
  mean = jnp.mean(z, axis=0, keepdims=True)
  zc = z - mean
  var = jnp.mean(zc * zc, axis=0, keepdims=True)
  zn = zc / jnp.sqrt(var + 1e-5)
  return jnp.maximum(zn * g + be, 0.0)


def _mlp_body(h_ref, a_ref, w1_ref, b1_ref, w2_ref, b2_ref, g_ref, be_ref,
              o_ref):
  o_ref[...] = _mlp(h_ref[...], a_ref[0, :_N], a_ref[1, :_N], w1_ref[...],
                    b1_ref[...], w2_ref[...], b2_ref[...], g_ref[...],
                    be_ref[...])


def _mlp_call(h, agg, w1, b1, w2, b2, gamma, beta):
  return pl.pallas_call(
      _mlp_body,
      out_shape=jax.ShapeDtypeStruct((_N, _HID), jnp.float32),
  )(h, agg, w1, b1.reshape(1, -1), w2, b2.reshape(1, -1),
    gamma.reshape(1, -1), beta.reshape(1, -1))


def _last_body(h2_ref, a_ref, w1_ref, b1_ref, w2_ref, b2_ref, g_ref, be_ref,
               h1_ref, b2d_ref, wj_ref, bj_ref, wc1_ref, bc1_ref,
               wc2_ref, bc2_ref, o_ref):
  h2 = h2_ref[...]
  h3 = _mlp(h2, a_ref[0, :_N], a_ref[1, :_N], w1_ref[...],
            b1_ref[...], w2_ref[...], b2_ref[...], g_ref[...], be_ref[...])
  hc = jnp.concatenate([h1_ref[...], h2, h3], axis=1)
  hj = jnp.dot(hc, wj_ref[...], preferred_element_type=jnp.float32) + bj_ref[...]
  gids = lax.broadcasted_iota(jnp.int32, (_NG, _N), 0)
  mask = (gids == b2d_ref[...]).astype(jnp.float32)
  pooled = jnp.dot(mask, hj, preferred_element_type=jnp.float32,
                   precision=lax.Precision.HIGHEST)
  cmid = jnp.maximum(
      jnp.dot(pooled, wc1_ref[...], preferred_element_type=jnp.float32)
      + bc1_ref[...], 0.0)
  o_ref[...] = (jnp.dot(cmid, wc2_ref[...], preferred_element_type=jnp.float32)
                + bc2_ref[...])


def _last_call(h2, agg, w1, b1, w2, b2, gamma, beta, h1, batch2d, wj, bj,
               wc1, bc1, wc2, bc2):
  return pl.pallas_call(
      _last_body,
      out_shape=jax.ShapeDtypeStruct((_NG, _NCLS), jnp.float32),
  )(h2, agg, w1, b1.reshape(1, -1), w2, b2.reshape(1, -1),
    gamma.reshape(1, -1), beta.reshape(1, -1), h1, batch2d, wj,
    bj.reshape(1, -1), wc1, bc1.reshape(1, -1), wc2, bc2.reshape(1, -1))


def kernel(x, edge_index, batch, params):
  batch2d = batch.reshape(1, _N)

  agg = _agg128(x, edge_index)
  h1 = _mlp_call(x, agg, params['W1_0'], params['b1_0'], params['W2_0'],
                 params['b2_0'], params['gamma_0'], params['beta_0'])
  agg = _agg64(h1, edge_index)
  h2 = _mlp_call(h1, agg, params['W1_1'], params['b1_1'], params['W2_1'],
                 params['b2_1'], params['gamma_1'], params['beta_1'])
  agg = _agg64(h2, edge_index)
  return _last_call(h2, agg, params['W1_2'], params['b1_2'], params['W2_2'],
                    params['b2_2'], params['gamma_2'], params['beta_2'],
                    h1, batch2d, params['Wj'], params['bj'],
                    params['Wc1'], params['bc1'], params['Wc2'],
                    params['bc2'])


# speedup vs baseline: 1.1189x; 1.0817x over previous
import functools

import jax
import jax.numpy as jnp
from jax import lax
from jax.experimental import pallas as pl
from jax.experimental.pallas import tpu as pltpu
from jax.experimental.pallas import tpu_sc as plsc

_N = 10000
_E = 320000
_F_IN = 128
_HID = 64
_NCLS = 2
_L = 3
_NG = 64

_SC_CORES = 2
_SC_TILES = 16
_NW = _SC_CORES * _SC_TILES
_EW = _E // _NW
_RPT = 632
_RPT_LAST = _N - 15 * _RPT
_CHUNK = 80
_IT = _EW // _CHUNK


def _make_agg(F, nbuf):
  chunk, it = _CHUNK, _IT
  full, tail = it // nbuf, it % nbuf
  mesh = plsc.VectorSubcoreMesh(core_axis_name="c", subcore_axis_name="s")

  @functools.partial(
      pl.kernel,
      out_type=jax.ShapeDtypeStruct((_SC_CORES, _N, F), jnp.float32),
      mesh=mesh,
      compiler_params=pltpu.CompilerParams(use_tc_tiling_on_sc=False),
      scratch_types=(
          [pltpu.VMEM((_EW,), jnp.int32)] * 2
          + [pltpu.VMEM((chunk, F), jnp.float32)] * nbuf
          + [pltpu.VMEM_SHARED((_N, F), jnp.float32)]
          + [pltpu.SemaphoreType.DMA] * nbuf
      ),
  )
  def agg(h_hbm, edge_hbm, out_hbm, src_v, dst_v, *rest):
    rows = rest[:nbuf]
    acc_sh = rest[nbuf]
    sems = rest[nbuf + 1:]
    c = lax.axis_index("c")
    s = lax.axis_index("s")
    w = c * _SC_TILES + s
    @pl.when(s != _SC_TILES - 1)
    def _():
      pltpu.sync_copy(h_hbm.at[pl.ds(s * _RPT, _RPT)],
                      acc_sh.at[pl.ds(s * _RPT, _RPT)])

    @pl.when(s == _SC_TILES - 1)
    def _():
      pltpu.sync_copy(h_hbm.at[pl.ds((_SC_TILES - 1) * _RPT, _RPT_LAST)],
                      acc_sh.at[pl.ds((_SC_TILES - 1) * _RPT, _RPT_LAST)])
    pltpu.sync_copy(edge_hbm.at[0, pl.ds(w * _EW, _EW)], src_v)
    pltpu.sync_copy(edge_hbm.at[1, pl.ds(w * _EW, _EW)], dst_v)
    plsc.subcore_barrier()

    def sidx(i):
      return src_v.at[pl.ds(i * chunk, chunk)]

    def didx(i):
      return dst_v.at[pl.ds(i * chunk, chunk)]

    for b in range(nbuf):
      pltpu.async_copy(h_hbm.at[sidx(b)], rows[b], sems[b])

    def body(j, carry):
      for k in range(nbuf):
        i = nbuf * j + k
        pltpu.make_async_copy(h_hbm.at[sidx(0)], rows[k], sems[k]).wait()
        pltpu.sync_copy(rows[k], acc_sh.at[didx(i)], add=True)
        nxt = lax.rem(i + nbuf, it)
        pltpu.async_copy(h_hbm.at[sidx(nxt)], rows[k], sems[k])
      return carry

    lax.fori_loop(0, full, body, 0)
    for k in range(nbuf):
      pltpu.make_async_copy(h_hbm.at[sidx(0)], rows[k], sems[k]).wait()
      if k < tail:
        pltpu.sync_copy(rows[k], acc_sh.at[didx(full * nbuf + k)], add=True)
    plsc.subcore_barrier()

    @pl.when(s != _SC_TILES - 1)
    def _():
      pltpu.sync_copy(acc_sh.at[pl.ds(s * _RPT, _RPT)],
                      out_hbm.at[c, pl.ds(s * _RPT, _RPT)])

    @pl.when(s == _SC_TILES - 1)
    def _():
      pltpu.sync_copy(acc_sh.at[pl.ds((_SC_TILES - 1) * _RPT, _RPT_LAST)],
                      out_hbm.at[c, pl.ds((_SC_TILES - 1) * _RPT, _RPT_LAST)])

  return agg


_agg128 = _make_agg(_F_IN, 3)
_agg64 = _make_agg(_HID, 8)


def _mlp(h, a0, a1, w1, b1, w2, b2, g, be):
  z = a0 + a1 - h
  z = jnp.dot(z, w1, preferred_element_type=jnp.float32) + b1
  z = jnp.maximum(z, 0.0)
  z = jnp.dot(z, w2, preferred_element_type=jnp.float32) + b2
  mean = jnp.mean(z, axis=0, keepdims=True)
  zc = z - mean
  var = jnp.mean(zc * zc, axis=0, keepdims=True)
  zn = zc / jnp.sqrt(var + 1e-5)
  return jnp.maximum(zn * g + be, 0.0)


def _mlp_body(h_ref, a_ref, w1_ref, b1_ref, w2_ref, b2_ref, g_ref, be_ref,
              o_ref):
  o_ref[...] = _mlp(h_ref[...], a_ref[0], a_ref[1], w1_ref[...],
                    b1_ref[...], w2_ref[...], b2_ref[...], g_ref[...],
                    be_ref[...])


def _mlp_call(h, agg, w1, b1, w2, b2, gamma, beta):
  return pl.pallas_call(
      _mlp_body,
      out_shape=jax.ShapeDtypeStruct((_N, _HID), jnp.float32),
  )(h, agg, w1, b1.reshape(1, -1), w2, b2.reshape(1, -1),
    gamma.reshape(1, -1), beta.reshape(1, -1))


def _last_body(h2_ref, a_ref, w1_ref, b1_ref, w2_ref, b2_ref, g_ref, be_ref,
               h1_ref, b2d_ref, wj_ref, bj_ref, wc1_ref, bc1_ref,
               wc2_ref, bc2_ref, o_ref):
  h2 = h2_ref[...]
  h3 = _mlp(h2, a_ref[0], a_ref[1], w1_ref[...],
            b1_ref[...], w2_ref[...], b2_ref[...], g_ref[...], be_ref[...])
  hc = jnp.concatenate([h1_ref[...], h2, h3], axis=1)
  hj = jnp.dot(hc, wj_ref[...], preferred_element_type=jnp.float32) + bj_ref[...]
  gids = lax.broadcasted_iota(jnp.int32, (_NG, _N), 0)
  mask = (gids == b2d_ref[...]).astype(jnp.float32)
  pooled = jnp.dot(mask, hj, preferred_element_type=jnp.float32,
                   precision=lax.Precision.HIGHEST)
  cmid = jnp.maximum(
      jnp.dot(pooled, wc1_ref[...], preferred_element_type=jnp.float32)
      + bc1_ref[...], 0.0)
  o_ref[...] = (jnp.dot(cmid, wc2_ref[...], preferred_element_type=jnp.float32)
                + bc2_ref[...])


def _last_call(h2, agg, w1, b1, w2, b2, gamma, beta, h1, batch2d, wj, bj,
               wc1, bc1, wc2, bc2):
  return pl.pallas_call(
      _last_body,
      out_shape=jax.ShapeDtypeStruct((_NG, _NCLS), jnp.float32),
  )(h2, agg, w1, b1.reshape(1, -1), w2, b2.reshape(1, -1),
    gamma.reshape(1, -1), beta.reshape(1, -1), h1, batch2d, wj,
    bj.reshape(1, -1), wc1, bc1.reshape(1, -1), wc2, bc2.reshape(1, -1))


def kernel(x, edge_index, batch, params):
  batch2d = batch.reshape(1, _N)

  agg = _agg128(x, edge_index)
  h1 = _mlp_call(x, agg, params['W1_0'], params['b1_0'], params['W2_0'],
                 params['b2_0'], params['gamma_0'], params['beta_0'])
  agg = _agg64(h1, edge_index)
  h2 = _mlp_call(h1, agg, params['W1_1'], params['b1_1'], params['W2_1'],
                 params['b2_1'], params['gamma_1'], params['beta_1'])
  agg = _agg64(h2, edge_index)
  return _last_call(h2, agg, params['W1_2'], params['b1_2'], params['W2_2'],
                    params['b2_2'], params['gamma_2'], params['beta_2'],
                    h1, batch2d, params['Wj'], params['bj'],
                    params['Wc1'], params['bc1'], params['Wc2'],
                    params['bc2'])
